# Initial kernel scaffold; baseline (speedup 1.0000x reference)
import jax
import jax.numpy as jnp
from jax.experimental import pallas as pl


def kernel(x, w1, b1, w2, b2):
    raise NotImplementedError("write your pallas kernel here")


# speedup vs baseline: 3.0444x; 3.0444x over previous
import functools

import jax
import jax.numpy as jnp
from jax.experimental import pallas as pl
from jax.experimental.pallas import tpu as pltpu


def _fused_kernel(p_ref, w1_ref, w2_ref, b1_ref, b2_ref, r1_ref, c2_ref,
                  *, Wp, W, H):
    s = pl.program_id(1)
    L = 8 * Wp
    S = 10 * Wp + 8
    base = pl.multiple_of(s * L, 8)

    p = p_ref[0, pl.ds(base, S), :]
    r1 = jnp.dot(p, w1_ref[...], preferred_element_type=jnp.float32)
    r1 = jnp.maximum(r1 + b1_ref[...], 0.0)

    i = jax.lax.broadcasted_iota(jnp.int32, (S, 1), 0)
    cp = i % Wp
    rp = s * 8 + i // Wp
    valid = (cp >= 1) & (cp <= W) & (rp >= 1) & (rp <= H)
    r1 = jnp.where(valid, r1, 0.0)

    r1v = r1[:10 * Wp].reshape(10, Wp, 64)[1:9, 1:W + 1, :]
    r1_ref[0] = jnp.transpose(r1v, (2, 0, 1))

    r1b = r1.astype(jnp.bfloat16)
    acc = None
    for ky in range(3):
        for kx in range(3):
            off = ky * Wp + kx
            contrib = jnp.dot(r1b[off:off + L, :], w2_ref[3 * ky + kx],
                              preferred_element_type=jnp.float32)
            acc = contrib if acc is None else acc + contrib
    out2 = acc + b2_ref[...]
    c2v = out2.reshape(8, Wp, 64)[:, :W, :]
    c2_ref[0] = jnp.transpose(c2v, (2, 0, 1))


def kernel(x, w1, b1, w2, b2):
    N, Cin, H, W = x.shape
    Cout = w1.shape[0]
    Wp = W + 2
    R = (H + 2) * Wp

    x_nhwc = jnp.transpose(x, (0, 2, 3, 1))
    xp = jnp.pad(x_nhwc, ((0, 0), (2, 2), (2, 2), (0, 0))).astype(jnp.bfloat16)
    taps = [xp[:, ky:ky + H + 2, kx:kx + W + 2, :]
            for ky in range(3) for kx in range(3)]
    patches = jnp.concatenate(taps, axis=-1).reshape(N, R, 9 * Cin)
    patches = jnp.pad(patches, ((0, 0), (0, 8), (0, 0)))
    R = R + 8

    w1f = jnp.transpose(w1, (2, 3, 1, 0)).reshape(9 * Cin, Cout)
    w1f = w1f.astype(jnp.bfloat16)
    w2t = jnp.transpose(w2, (2, 3, 1, 0)).reshape(9, Cout, Cout)
    w2t = w2t.astype(jnp.bfloat16)
    b1r = b1.reshape(1, Cout)
    b2r = b2.reshape(1, Cout)

    kern = functools.partial(_fused_kernel, Wp=Wp, W=W, H=H)
    r1, c2 = pl.pallas_call(
        kern,
        out_shape=(
            jax.ShapeDtypeStruct((N, Cout, H, W), jnp.float32),
            jax.ShapeDtypeStruct((N, Cout, H, W), jnp.float32),
        ),
        grid_spec=pltpu.PrefetchScalarGridSpec(
            num_scalar_prefetch=0,
            grid=(N, H // 8),
            in_specs=[
                pl.BlockSpec((1, R, 9 * Cin), lambda n, s: (n, 0, 0)),
                pl.BlockSpec((9 * Cin, Cout), lambda n, s: (0, 0)),
                pl.BlockSpec((9, Cout, Cout), lambda n, s: (0, 0, 0)),
                pl.BlockSpec((1, Cout), lambda n, s: (0, 0)),
                pl.BlockSpec((1, Cout), lambda n, s: (0, 0)),
            ],
            out_specs=(
                pl.BlockSpec((1, Cout, 8, W), lambda n, s: (n, 0, s, 0)),
                pl.BlockSpec((1, Cout, 8, W), lambda n, s: (n, 0, s, 0)),
            ),
        ),
        compiler_params=pltpu.CompilerParams(
            dimension_semantics=("parallel", "arbitrary")),
    )(patches, w1f, w2t, b1r, b2r)

    return [r1, r1, c2]


# speedup vs baseline: 8.2450x; 2.7082x over previous
import functools

import jax
import jax.numpy as jnp
from jax.experimental import pallas as pl
from jax.experimental.pallas import tpu as pltpu

_RS = 256


def _fused_kernel(p_ref, w1_ref, w2_ref, b1_ref, b2_ref, r1_ref, c2_ref,
                  *, W, H):
    s = pl.program_id(1)
    L = 8 * _RS
    S = 10 * _RS + 128
    base = pl.multiple_of(s * L, 128)

    p = p_ref[0, :, pl.ds(base, S)]
    r1 = jnp.dot(w1_ref[...], p, preferred_element_type=jnp.float32)
    r1 = jnp.maximum(r1 + b1_ref[...], 0.0)

    lane = jax.lax.broadcasted_iota(jnp.int32, (1, S), 1)
    cp = lane & (_RS - 1)
    rp = s * 8 + (lane >> 8)
    valid = (cp >= 1) & (cp <= W) & (rp >= 1) & (rp <= H)
    r1 = jnp.where(valid, r1, 0.0)

    r1_ref[0] = r1[:, _RS + 1:9 * _RS + 1].reshape(64, 8, _RS)[:, :, :W]

    r1b = r1.astype(jnp.bfloat16)
    acc = None
    for kx in range(3):
        tap = jnp.concatenate(
            [r1b[:, ky * _RS + kx:ky * _RS + kx + L] for ky in range(3)],
            axis=0)
        contrib = jnp.dot(w2_ref[kx], tap,
                          preferred_element_type=jnp.float32)
        acc = contrib if acc is None else acc + contrib
    out2 = acc + b2_ref[...]
    c2_ref[0] = out2.reshape(64, 8, _RS)[:, :, :W]


def kernel(x, w1, b1, w2, b2):
    N, Cin, H, W = x.shape
    Cout = w1.shape[0]
    nstrip = H // 8
    LT = (H + 2) * _RS + 3 * _RS

    xp = jnp.pad(x, ((0, 0), (0, 0), (2, 2), (2, _RS + 2 - W - 2)))
    xp = xp.astype(jnp.bfloat16)
    taps = [xp[:, :, ky:ky + H + 2, kx:kx + _RS]
            for ky in range(3) for kx in range(3)]
    patches = jnp.concatenate(taps, axis=1).reshape(N, 9 * Cin, (H + 2) * _RS)
    patches = jnp.pad(patches, ((0, 0), (0, 0), (0, LT - (H + 2) * _RS)))

    w1t = jnp.transpose(w1, (0, 2, 3, 1)).reshape(Cout, 9 * Cin)
    w1t = w1t.astype(jnp.bfloat16)
    w2t4 = jnp.transpose(w2, (2, 3, 0, 1))
    w2p = jnp.stack([
        jnp.concatenate([w2t4[0, kx], w2t4[1, kx], w2t4[2, kx]], axis=1)
        for kx in range(3)])
    w2p = w2p.astype(jnp.bfloat16)
    b1c = b1.reshape(Cout, 1)
    b2c = b2.reshape(Cout, 1)

    kern = functools.partial(_fused_kernel, W=W, H=H)
    r1, c2 = pl.pallas_call(
        kern,
        out_shape=(
            jax.ShapeDtypeStruct((N, Cout, H, W), jnp.float32),
            jax.ShapeDtypeStruct((N, Cout, H, W), jnp.float32),
        ),
        grid_spec=pltpu.PrefetchScalarGridSpec(
            num_scalar_prefetch=0,
            grid=(N, nstrip),
            in_specs=[
                pl.BlockSpec((1, 9 * Cin, LT), lambda n, s: (n, 0, 0)),
                pl.BlockSpec((Cout, 9 * Cin), lambda n, s: (0, 0)),
                pl.BlockSpec((3, Cout, 3 * Cout), lambda n, s: (0, 0, 0)),
                pl.BlockSpec((Cout, 1), lambda n, s: (0, 0)),
                pl.BlockSpec((Cout, 1), lambda n, s: (0, 0)),
            ],
            out_specs=(
                pl.BlockSpec((1, Cout, 8, W), lambda n, s: (n, 0, s, 0)),
                pl.BlockSpec((1, Cout, 8, W), lambda n, s: (n, 0, s, 0)),
            ),
        ),
        compiler_params=pltpu.CompilerParams(
            dimension_semantics=("parallel", "arbitrary")),
    )(patches, w1t, w2p, b1c, b2c)

    return [r1, r1, c2]


# speedup vs baseline: 9.9721x; 1.2095x over previous
import functools

import jax
import jax.numpy as jnp
from jax.experimental import pallas as pl
from jax.experimental.pallas import tpu as pltpu

_RS = 256
_ST = 32


def _fused_kernel(p_ref, w1_ref, w2_ref, b1_ref, b2_ref, r1_ref, c2_ref,
                  *, W, H, st):
    s = pl.program_id(1)
    L = st * _RS
    S = (st + 2) * _RS + 128
    base = pl.multiple_of(s * L, 128)

    p = p_ref[0, :, pl.ds(base, S)]
    r1 = jnp.dot(w1_ref[...], p, preferred_element_type=jnp.float32)
    r1 = jnp.maximum(r1 + b1_ref[...], 0.0)

    lane = jax.lax.broadcasted_iota(jnp.int32, (1, S), 1)
    cp = lane & (_RS - 1)
    rp = s * st + (lane >> 8)
    valid = (cp >= 1) & (cp <= W) & (rp >= 1) & (rp <= H)
    r1 = jnp.where(valid, r1, 0.0)

    r1_ref[0] = (r1[:, _RS + 1:(st + 1) * _RS + 1]
                 .reshape(64, st, _RS)[:, :, :W])

    r1b = r1.astype(jnp.bfloat16)
    acc = None
    for kx in range(3):
        tap = jnp.concatenate(
            [r1b[:, ky * _RS + kx:ky * _RS + kx + L] for ky in range(3)],
            axis=0)
        contrib = jnp.dot(w2_ref[kx], tap,
                          preferred_element_type=jnp.float32)
        acc = contrib if acc is None else acc + contrib
    out2 = acc + b2_ref[...]
    c2_ref[0] = out2.reshape(64, st, _RS)[:, :, :W]


def kernel(x, w1, b1, w2, b2):
    N, Cin, H, W = x.shape
    Cout = w1.shape[0]
    st = _ST if H % _ST == 0 else 8
    nstrip = H // st
    LT = (H + 2) * _RS + 3 * _RS

    xp = jnp.pad(x, ((0, 0), (0, 0), (2, 2), (2, _RS + 2 - W - 2)))
    xp = xp.astype(jnp.bfloat16)
    taps = [xp[:, :, ky:ky + H + 2, kx:kx + _RS]
            for ky in range(3) for kx in range(3)]
    patches = jnp.concatenate(taps, axis=1).reshape(N, 9 * Cin, (H + 2) * _RS)
    patches = jnp.pad(patches, ((0, 0), (0, 0), (0, LT - (H + 2) * _RS)))

    w1t = jnp.transpose(w1, (0, 2, 3, 1)).reshape(Cout, 9 * Cin)
    w1t = w1t.astype(jnp.bfloat16)
    w2t4 = jnp.transpose(w2, (2, 3, 0, 1))
    w2p = jnp.stack([
        jnp.concatenate([w2t4[0, kx], w2t4[1, kx], w2t4[2, kx]], axis=1)
        for kx in range(3)])
    w2p = w2p.astype(jnp.bfloat16)
    b1c = b1.reshape(Cout, 1)
    b2c = b2.reshape(Cout, 1)

    kern = functools.partial(_fused_kernel, W=W, H=H, st=st)
    r1, c2 = pl.pallas_call(
        kern,
        out_shape=(
            jax.ShapeDtypeStruct((N, Cout, H, W), jnp.float32),
            jax.ShapeDtypeStruct((N, Cout, H, W), jnp.float32),
        ),
        grid_spec=pltpu.PrefetchScalarGridSpec(
            num_scalar_prefetch=0,
            grid=(N, nstrip),
            in_specs=[
                pl.BlockSpec((1, 9 * Cin, LT), lambda n, s: (n, 0, 0)),
                pl.BlockSpec((Cout, 9 * Cin), lambda n, s: (0, 0)),
                pl.BlockSpec((3, Cout, 3 * Cout), lambda n, s: (0, 0, 0)),
                pl.BlockSpec((Cout, 1), lambda n, s: (0, 0)),
                pl.BlockSpec((Cout, 1), lambda n, s: (0, 0)),
            ],
            out_specs=(
                pl.BlockSpec((1, Cout, st, W), lambda n, s: (n, 0, s, 0)),
                pl.BlockSpec((1, Cout, st, W), lambda n, s: (n, 0, s, 0)),
            ),
        ),
        compiler_params=pltpu.CompilerParams(
            dimension_semantics=("parallel", "arbitrary")),
    )(patches, w1t, w2p, b1c, b2c)

    return [r1, r1, c2]


# speedup vs baseline: 10.4037x; 1.0433x over previous
import functools

import jax
import jax.numpy as jnp
from jax.experimental import pallas as pl
from jax.experimental.pallas import tpu as pltpu

_RS = 256
_ST = 56


def _fused_kernel(p_ref, w1_ref, w2_ref, b1_ref, b2_ref, r1_ref, c2_ref,
                  *, W, H, st):
    s = pl.program_id(1)
    L = st * _RS
    S = (st + 2) * _RS + 128
    base = pl.multiple_of(s * L, 128)

    p = p_ref[0, :, pl.ds(base, S)]
    r1 = jnp.dot(w1_ref[...], p, preferred_element_type=jnp.float32)
    r1 = jnp.maximum(r1 + b1_ref[...], 0.0)

    lane = jax.lax.broadcasted_iota(jnp.int32, (1, S), 1)
    cp = lane & (_RS - 1)
    rp = s * st + (lane >> 8)
    valid = (cp >= 1) & (cp <= W) & (rp >= 1) & (rp <= H)
    r1 = jnp.where(valid, r1, 0.0)

    r1_ref[0] = (r1[:, _RS + 1:(st + 1) * _RS + 1]
                 .reshape(64, st, _RS)[:, :, :W])

    r1b = r1.astype(jnp.bfloat16)
    acc = None
    for kx in range(3):
        tap = jnp.concatenate(
            [r1b[:, ky * _RS + kx:ky * _RS + kx + L] for ky in range(3)],
            axis=0)
        contrib = jnp.dot(w2_ref[kx], tap,
                          preferred_element_type=jnp.float32)
        acc = contrib if acc is None else acc + contrib
    out2 = acc + b2_ref[...]
    c2_ref[0] = out2.reshape(64, st, _RS)[:, :, :W]


def kernel(x, w1, b1, w2, b2):
    N, Cin, H, W = x.shape
    Cout = w1.shape[0]
    st = _ST if H % _ST == 0 else 8
    nstrip = H // st
    LT = (H + 2) * _RS + 3 * _RS

    xp = jnp.pad(x, ((0, 0), (0, 0), (2, 2), (2, _RS + 2 - W - 2)))
    xp = xp.astype(jnp.bfloat16)
    taps = [xp[:, :, ky:ky + H + 2, kx:kx + _RS]
            for ky in range(3) for kx in range(3)]
    patches = jnp.concatenate(taps, axis=1).reshape(N, 9 * Cin, (H + 2) * _RS)
    patches = jnp.pad(patches, ((0, 0), (0, 0), (0, LT - (H + 2) * _RS)))

    w1t = jnp.transpose(w1, (0, 2, 3, 1)).reshape(Cout, 9 * Cin)
    w1t = w1t.astype(jnp.bfloat16)
    w2t4 = jnp.transpose(w2, (2, 3, 0, 1))
    w2p = jnp.stack([
        jnp.concatenate([w2t4[0, kx], w2t4[1, kx], w2t4[2, kx]], axis=1)
        for kx in range(3)])
    w2p = w2p.astype(jnp.bfloat16)
    b1c = b1.reshape(Cout, 1)
    b2c = b2.reshape(Cout, 1)

    kern = functools.partial(_fused_kernel, W=W, H=H, st=st)
    r1, c2 = pl.pallas_call(
        kern,
        out_shape=(
            jax.ShapeDtypeStruct((N, Cout, H, W), jnp.float32),
            jax.ShapeDtypeStruct((N, Cout, H, W), jnp.float32),
        ),
        grid_spec=pltpu.PrefetchScalarGridSpec(
            num_scalar_prefetch=0,
            grid=(N, nstrip),
            in_specs=[
                pl.BlockSpec((1, 9 * Cin, LT), lambda n, s: (n, 0, 0)),
                pl.BlockSpec((Cout, 9 * Cin), lambda n, s: (0, 0)),
                pl.BlockSpec((3, Cout, 3 * Cout), lambda n, s: (0, 0, 0)),
                pl.BlockSpec((Cout, 1), lambda n, s: (0, 0)),
                pl.BlockSpec((Cout, 1), lambda n, s: (0, 0)),
            ],
            out_specs=(
                pl.BlockSpec((1, Cout, st, W), lambda n, s: (n, 0, s, 0)),
                pl.BlockSpec((1, Cout, st, W), lambda n, s: (n, 0, s, 0)),
            ),
        ),
        compiler_params=pltpu.CompilerParams(
            dimension_semantics=("parallel", "arbitrary")),
    )(patches, w1t, w2p, b1c, b2c)

    return [r1, r1, c2]
